# baseline (speedup 1.0000x reference)
import numpy as np
import jax
import jax.numpy as jnp
from jax.experimental import pallas as pl
from jax.experimental.pallas import tpu as pltpu

_B, _T, _C, _H, _W = 4, 4, 8, 224, 224
_R = _B * _T
_N = _H * _W
_K = _N // 2
_DENOM = float(_R * _K * _C)


def _np_threefry2x32(k0, k1, x0, x1):
    def rotl(x, d):
        return ((x << np.uint32(d)) | (x >> np.uint32(32 - d))).astype(np.uint32)

    ks0, ks1 = np.uint32(k0), np.uint32(k1)
    ks2 = np.uint32(ks0 ^ ks1 ^ np.uint32(0x1BD11BDA))
    ks = [ks0, ks1, ks2]
    rotations = [(13, 15, 26, 6), (17, 29, 16, 24)]
    x0 = (x0 + ks0).astype(np.uint32)
    x1 = (x1 + ks1).astype(np.uint32)
    for i in range(5):
        for r in rotations[i % 2]:
            x0 = (x0 + x1).astype(np.uint32)
            x1 = rotl(x1, r)
            x1 = (x1 ^ x0).astype(np.uint32)
        x0 = (x0 + ks[(i + 1) % 3]).astype(np.uint32)
        x1 = (x1 + ks[(i + 2) % 3] + np.uint32(i + 1)).astype(np.uint32)
    return x0, x1


def _np_uniform_table(seed, size):
    k0 = np.uint32(np.uint64(seed) >> np.uint64(32))
    k1 = np.uint32(np.uint64(seed) & np.uint64(0xFFFFFFFF))
    lo = np.arange(size, dtype=np.uint32)
    hi = np.zeros(size, dtype=np.uint32)
    o0, o1 = _np_threefry2x32(k0, k1, hi, lo)
    bits = o0 ^ o1
    fb = (bits >> np.uint32(9)) | np.uint32(0x3F800000)
    floats = fb.view(np.float32) - np.float32(1.0)
    tiny = np.float32(np.finfo(np.float32).tiny)
    return np.maximum(tiny, floats * (np.float32(1.0) - tiny) + tiny)


_E = (np.float32(-1.0) / np.log(_np_uniform_table(42, _R * _N))).reshape(
    _R, _H, _W)

_INT_MAX = np.int32(2147483647)


def _mae_body(t_ref, p_ref, g_ref, o_ref, key_ref, d_ref):
    r = pl.program_id(0)
    t = t_ref[0]
    p = p_ref[0]
    g = g_ref[0]

    v = (jnp.sqrt(jnp.sum(t * t, axis=0)) + 0.5) * g
    d = jnp.sum(jnp.abs(p - t), axis=0)

    key_ref[r] = jax.lax.bitcast_convert_type(v, jnp.int32)
    d_ref[r] = d

    @pl.when(r == _R - 1)
    def _phase_b():
        key = key_ref[...]
        dd = d_ref[...]

        def body(_, lohi):
            lo, hi = lohi
            mid = (lo + hi) >> 1
            cnt = jnp.sum((key >= mid).astype(jnp.int32), axis=(1, 2),
                          keepdims=True)
            pred = cnt >= _K
            return jnp.where(pred, mid, lo), jnp.where(pred, hi, mid)

        lo0 = jnp.zeros((_R, 1, 1), jnp.int32)
        hi0 = jnp.full((_R, 1, 1), _INT_MAX, jnp.int32)
        tau, _ = jax.lax.fori_loop(0, 31, body, (lo0, hi0))

        mask_ge = key >= tau
        count_ge = jnp.sum(mask_ge.astype(jnp.int32), axis=(1, 2),
                           keepdims=True)
        sum_ge = jnp.sum(jnp.where(mask_ge, dd, 0.0), axis=(1, 2),
                         keepdims=True)
        exact = jnp.all(count_ge == _K)

        @pl.when(exact)
        def _no_ties():
            o_ref[0, 0] = jnp.sum(sum_ge) * (1.0 / _DENOM)

        @pl.when(jnp.logical_not(exact))
        def _ties():
            mask_eq = key == tau
            count_eq = jnp.sum(mask_eq.astype(jnp.float32), axis=(1, 2),
                               keepdims=True)
            sum_eq = jnp.sum(jnp.where(mask_eq, dd, 0.0), axis=(1, 2),
                             keepdims=True)
            count_gt = count_ge.astype(jnp.float32) - count_eq
            sum_gt = sum_ge - sum_eq
            need = jnp.float32(_K) - count_gt
            total = jnp.sum(sum_gt + need * sum_eq / count_eq)
            o_ref[0, 0] = total * (1.0 / _DENOM)


def kernel(out_preds, out_targets, tl, tv, x_rep, in_x, in_l, in_v, in_n):
    t = out_targets.reshape(_R, _C, _H, _W)
    p = out_preds.reshape(_R, _C, _H, _W)
    g = jnp.asarray(_E)
    out = pl.pallas_call(
        _mae_body,
        grid=(_R,),
        in_specs=[
            pl.BlockSpec((1, _C, _H, _W), lambda r: (r, 0, 0, 0)),
            pl.BlockSpec((1, _C, _H, _W), lambda r: (r, 0, 0, 0)),
            pl.BlockSpec((1, _H, _W), lambda r: (r, 0, 0)),
        ],
        out_specs=pl.BlockSpec((1, 1), lambda r: (0, 0), memory_space=pltpu.SMEM),
        out_shape=jax.ShapeDtypeStruct((1, 1), jnp.float32),
        scratch_shapes=[
            pltpu.VMEM((_R, _H, _W), jnp.int32),
            pltpu.VMEM((_R, _H, _W), jnp.float32),
        ],
    )(t, p, g)
    return out[0, 0]


# speedup vs baseline: 1.0206x; 1.0206x over previous
import numpy as np
import jax
import jax.numpy as jnp
from jax.experimental import pallas as pl
from jax.experimental.pallas import tpu as pltpu

_B, _T, _C, _H, _W = 4, 4, 8, 224, 224
_R = _B * _T
_N = _H * _W
_K = _N // 2
_DENOM = float(_R * _K * _C)


def _np_threefry2x32(k0, k1, x0, x1):
    def rotl(x, d):
        return ((x << np.uint32(d)) | (x >> np.uint32(32 - d))).astype(np.uint32)

    ks0, ks1 = np.uint32(k0), np.uint32(k1)
    ks2 = np.uint32(ks0 ^ ks1 ^ np.uint32(0x1BD11BDA))
    ks = [ks0, ks1, ks2]
    rotations = [(13, 15, 26, 6), (17, 29, 16, 24)]
    x0 = (x0 + ks0).astype(np.uint32)
    x1 = (x1 + ks1).astype(np.uint32)
    for i in range(5):
        for r in rotations[i % 2]:
            x0 = (x0 + x1).astype(np.uint32)
            x1 = rotl(x1, r)
            x1 = (x1 ^ x0).astype(np.uint32)
        x0 = (x0 + ks[(i + 1) % 3]).astype(np.uint32)
        x1 = (x1 + ks[(i + 2) % 3] + np.uint32(i + 1)).astype(np.uint32)
    return x0, x1


def _np_uniform_table(seed, size):
    k0 = np.uint32(np.uint64(seed) >> np.uint64(32))
    k1 = np.uint32(np.uint64(seed) & np.uint64(0xFFFFFFFF))
    lo = np.arange(size, dtype=np.uint32)
    hi = np.zeros(size, dtype=np.uint32)
    o0, o1 = _np_threefry2x32(k0, k1, hi, lo)
    bits = o0 ^ o1
    fb = (bits >> np.uint32(9)) | np.uint32(0x3F800000)
    floats = fb.view(np.float32) - np.float32(1.0)
    tiny = np.float32(np.finfo(np.float32).tiny)
    return np.maximum(tiny, floats * (np.float32(1.0) - tiny) + tiny)


_E = (np.float32(-1.0) / np.log(_np_uniform_table(42, _R * _N))).reshape(
    _R, _H, _W)

_INT_MAX = np.int32(2147483647)


def _mae_body(t_ref, p_ref, g_ref, o_ref, key_ref, d_ref):
    r = pl.program_id(0)
    t = t_ref[0]
    p = p_ref[0]
    g = g_ref[0]

    v = (jnp.sqrt(jnp.sum(t * t, axis=0)) + 0.5) * g
    d = jnp.sum(jnp.abs(p - t), axis=0)

    key_ref[r] = jax.lax.bitcast_convert_type(v, jnp.int32)
    d_ref[r] = d

    @pl.when(r == _R - 1)
    def _phase_b():
        key = key_ref[...]
        dd = d_ref[...]

        def body(_, lohi):
            lo, hi = lohi
            mid = lo + ((hi - lo) >> 1)
            cnt = jnp.sum((key >= mid).astype(jnp.int32), axis=(1, 2),
                          keepdims=True)
            pred = cnt >= _K
            return jnp.where(pred, mid, lo), jnp.where(pred, hi, mid)

        lo0 = jnp.zeros((_R, 1, 1), jnp.int32)
        hi0 = jnp.full((_R, 1, 1), _INT_MAX, jnp.int32)
        tau, _ = jax.lax.fori_loop(0, 31, body, (lo0, hi0))

        mask_ge = key >= tau
        count_ge = jnp.sum(mask_ge.astype(jnp.int32), axis=(1, 2),
                           keepdims=True)
        sum_ge = jnp.sum(jnp.where(mask_ge, dd, 0.0), axis=(1, 2),
                         keepdims=True)
        exact = jnp.all(count_ge == _K)

        @pl.when(exact)
        def _no_ties():
            o_ref[0, 0] = jnp.sum(sum_ge) * (1.0 / _DENOM)

        @pl.when(jnp.logical_not(exact))
        def _ties():
            mask_eq = key == tau
            count_eq = jnp.sum(mask_eq.astype(jnp.float32), axis=(1, 2),
                               keepdims=True)
            sum_eq = jnp.sum(jnp.where(mask_eq, dd, 0.0), axis=(1, 2),
                             keepdims=True)
            count_gt = count_ge.astype(jnp.float32) - count_eq
            sum_gt = sum_ge - sum_eq
            need = jnp.float32(_K) - count_gt
            total = jnp.sum(sum_gt + need * sum_eq / count_eq)
            o_ref[0, 0] = total * (1.0 / _DENOM)


def kernel(out_preds, out_targets, tl, tv, x_rep, in_x, in_l, in_v, in_n):
    t = out_targets.reshape(_R, _C, _H, _W)
    p = out_preds.reshape(_R, _C, _H, _W)
    g = jnp.asarray(_E)
    out = pl.pallas_call(
        _mae_body,
        grid=(_R,),
        in_specs=[
            pl.BlockSpec((1, _C, _H, _W), lambda r: (r, 0, 0, 0)),
            pl.BlockSpec((1, _C, _H, _W), lambda r: (r, 0, 0, 0)),
            pl.BlockSpec((1, _H, _W), lambda r: (r, 0, 0)),
        ],
        out_specs=pl.BlockSpec((1, 1), lambda r: (0, 0), memory_space=pltpu.SMEM),
        out_shape=jax.ShapeDtypeStruct((1, 1), jnp.float32),
        scratch_shapes=[
            pltpu.VMEM((_R, _H, _W), jnp.int32),
            pltpu.VMEM((_R, _H, _W), jnp.float32),
        ],
    )(t, p, g)
    return out[0, 0]


# speedup vs baseline: 1.0291x; 1.0083x over previous
import numpy as np
import jax
import jax.numpy as jnp
from jax.experimental import pallas as pl
from jax.experimental.pallas import tpu as pltpu

_B, _T, _C, _H, _W = 4, 4, 8, 224, 224
_R = _B * _T
_N = _H * _W
_K = _N // 2
_DENOM = float(_R * _K * _C)


def _np_threefry2x32(k0, k1, x0, x1):
    def rotl(x, d):
        return ((x << np.uint32(d)) | (x >> np.uint32(32 - d))).astype(np.uint32)

    ks0, ks1 = np.uint32(k0), np.uint32(k1)
    ks2 = np.uint32(ks0 ^ ks1 ^ np.uint32(0x1BD11BDA))
    ks = [ks0, ks1, ks2]
    rotations = [(13, 15, 26, 6), (17, 29, 16, 24)]
    x0 = (x0 + ks0).astype(np.uint32)
    x1 = (x1 + ks1).astype(np.uint32)
    for i in range(5):
        for r in rotations[i % 2]:
            x0 = (x0 + x1).astype(np.uint32)
            x1 = rotl(x1, r)
            x1 = (x1 ^ x0).astype(np.uint32)
        x0 = (x0 + ks[(i + 1) % 3]).astype(np.uint32)
        x1 = (x1 + ks[(i + 2) % 3] + np.uint32(i + 1)).astype(np.uint32)
    return x0, x1


def _np_uniform_table(seed, size):
    k0 = np.uint32(np.uint64(seed) >> np.uint64(32))
    k1 = np.uint32(np.uint64(seed) & np.uint64(0xFFFFFFFF))
    lo = np.arange(size, dtype=np.uint32)
    hi = np.zeros(size, dtype=np.uint32)
    o0, o1 = _np_threefry2x32(k0, k1, hi, lo)
    bits = o0 ^ o1
    fb = (bits >> np.uint32(9)) | np.uint32(0x3F800000)
    floats = fb.view(np.float32) - np.float32(1.0)
    tiny = np.float32(np.finfo(np.float32).tiny)
    return np.maximum(tiny, floats * (np.float32(1.0) - tiny) + tiny)


_E = (np.float32(-1.0) / np.log(_np_uniform_table(42, _R * _N))).reshape(
    _R, _H, _W)

_INT_MAX = np.int32(2147483647)


def _mae_body(t_ref, p_ref, g_ref, o_ref, key_ref, d_ref):
    r = pl.program_id(0)
    g = g_ref[0]

    t0 = t_ref[0, 0]
    p0 = p_ref[0, 0]
    s2 = t0 * t0
    d = jnp.abs(p0 - t0)
    for c in range(1, _C):
        tc = t_ref[0, c]
        pc = p_ref[0, c]
        s2 = s2 + tc * tc
        d = d + jnp.abs(pc - tc)

    v = (jnp.sqrt(s2) + 0.5) * g

    key_ref[r] = jax.lax.bitcast_convert_type(v, jnp.int32)
    d_ref[r] = d

    @pl.when(r == _R - 1)
    def _phase_b():
        key = key_ref[...]
        dd = d_ref[...]

        def body(_, lohi):
            lo, hi = lohi
            mid = lo + ((hi - lo) >> 1)
            cnt = jnp.sum((key >= mid).astype(jnp.int32), axis=(1, 2),
                          keepdims=True)
            pred = cnt >= _K
            return jnp.where(pred, mid, lo), jnp.where(pred, hi, mid)

        lo0 = jnp.zeros((_R, 1, 1), jnp.int32)
        hi0 = jnp.full((_R, 1, 1), _INT_MAX, jnp.int32)
        tau, _ = jax.lax.fori_loop(0, 31, body, (lo0, hi0))

        mask_ge = key >= tau
        count_ge = jnp.sum(mask_ge.astype(jnp.int32), axis=(1, 2),
                           keepdims=True)
        sum_ge = jnp.sum(jnp.where(mask_ge, dd, 0.0), axis=(1, 2),
                         keepdims=True)
        exact = jnp.all(count_ge == _K)

        @pl.when(exact)
        def _no_ties():
            o_ref[0, 0] = jnp.sum(sum_ge) * (1.0 / _DENOM)

        @pl.when(jnp.logical_not(exact))
        def _ties():
            mask_eq = key == tau
            count_eq = jnp.sum(mask_eq.astype(jnp.float32), axis=(1, 2),
                               keepdims=True)
            sum_eq = jnp.sum(jnp.where(mask_eq, dd, 0.0), axis=(1, 2),
                             keepdims=True)
            count_gt = count_ge.astype(jnp.float32) - count_eq
            sum_gt = sum_ge - sum_eq
            need = jnp.float32(_K) - count_gt
            total = jnp.sum(sum_gt + need * sum_eq / count_eq)
            o_ref[0, 0] = total * (1.0 / _DENOM)


def kernel(out_preds, out_targets, tl, tv, x_rep, in_x, in_l, in_v, in_n):
    t = out_targets.reshape(_R, _C, _H, _W)
    p = out_preds.reshape(_R, _C, _H, _W)
    g = jnp.asarray(_E)
    out = pl.pallas_call(
        _mae_body,
        grid=(_R,),
        in_specs=[
            pl.BlockSpec((1, _C, _H, _W), lambda r: (r, 0, 0, 0)),
            pl.BlockSpec((1, _C, _H, _W), lambda r: (r, 0, 0, 0)),
            pl.BlockSpec((1, _H, _W), lambda r: (r, 0, 0)),
        ],
        out_specs=pl.BlockSpec((1, 1), lambda r: (0, 0), memory_space=pltpu.SMEM),
        out_shape=jax.ShapeDtypeStruct((1, 1), jnp.float32),
        scratch_shapes=[
            pltpu.VMEM((_R, _H, _W), jnp.int32),
            pltpu.VMEM((_R, _H, _W), jnp.float32),
        ],
    )(t, p, g)
    return out[0, 0]
